# Initial kernel scaffold; baseline (speedup 1.0000x reference)
import jax
import jax.numpy as jnp
from jax.experimental import pallas as pl


def kernel(seq, adj, conv_weight):
    raise NotImplementedError("write your pallas kernel here")


# speedup vs baseline: 1.7285x; 1.7285x over previous
import jax
import jax.numpy as jnp
from jax.experimental import pallas as pl
from jax.experimental.pallas import tpu as pltpu

_C = 128
_N = 4096
_BR = 256


def _fused_step(wcat_ref, seq_ref, adj2_ref, out_ref, scat_ref):
    i = pl.program_id(0)

    @pl.when(i == 0)
    def _init():
        spre = jnp.dot(wcat_ref[...], seq_ref[...],
                       preferred_element_type=jnp.float32)
        scat_ref[:, :_N] = spre[:_C, :]
        scat_ref[:, _N:] = spre[_C:, :]
        out_ref[...] = jnp.zeros_like(out_ref)

    adj2_blk = adj2_ref[...]
    y = jax.lax.dot_general(scat_ref[...], adj2_blk,
                            (((1,), (1,)), ((), ())),
                            preferred_element_type=jnp.float32)
    out_ref[...] += jnp.dot(y, adj2_blk[:, :_N],
                            preferred_element_type=jnp.float32)


def kernel(seq, adj, conv_weight):
    n = adj.shape[0]
    adj2 = adj.reshape(n // 2, 2 * n)
    wcat = conv_weight.transpose(2, 0, 1).reshape(2 * _C, _C)
    grid = (adj2.shape[0] // _BR,)
    return pl.pallas_call(
        _fused_step,
        grid=grid,
        in_specs=[
            pl.BlockSpec((2 * _C, _C), lambda i: (0, 0)),
            pl.BlockSpec((_C, _N), lambda i: (0, 0)),
            pl.BlockSpec((_BR, 2 * _N), lambda i: (i, 0)),
        ],
        out_specs=pl.BlockSpec((_C, _N), lambda i: (0, 0)),
        out_shape=jax.ShapeDtypeStruct((_C, _N), jnp.float32),
        scratch_shapes=[pltpu.VMEM((_C, 2 * _N), jnp.float32)],
    )(wcat, seq, adj2)
